# baseline (speedup 1.0000x reference)
import functools

import jax
import jax.numpy as jnp
from jax import lax
from jax.experimental import pallas as pl
from jax.experimental.pallas import tpu as pltpu
from jax.experimental.pallas import tpu_sc as plsc

MAXLEN = 2048
D_MODEL = 1024

NC = 2
NS = 16
NW = NC * NS
CH = 16
CHW = CH * D_MODEL
NBUF = 7
PF = 3


def _sc_kernel_body(B, x_hbm, pos_hbm, out_hbm, *refs):
    bufs = refs[:NBUF]
    lsems = refs[NBUF:2 * NBUF]
    ssems = refs[2 * NBUF:3 * NBUF]

    rows_total = B * MAXLEN
    rpw = rows_total // NW
    n_chunk = rpw // CH

    wid = lax.axis_index("s") * NC + lax.axis_index("c")
    base = wid * rpw * D_MODEL

    def start_load(c):
        p = c % NBUF
        return pltpu.async_copy(
            x_hbm.at[pl.ds(base + c * CHW, CHW)], bufs[p], lsems[p])

    loads = {}
    stores = {}
    for c in range(min(PF, n_chunk)):
        loads[c] = start_load(c)
    for c in range(n_chunk):
        p = c % NBUF
        loads.pop(c).wait()
        stores[c] = pltpu.async_copy(
            bufs[p], out_hbm.at[pl.ds(base + c * CHW, CHW)], ssems[p])
        nxt = c + PF
        if nxt < n_chunk:
            prev = nxt - NBUF
            if prev >= 0:
                stores.pop(prev).wait()
            loads[nxt] = start_load(nxt)
    for st in stores.values():
        st.wait()


def _make_sc_call(B):
    mesh = plsc.VectorSubcoreMesh(core_axis_name="c", subcore_axis_name="s")
    return pl.kernel(
        functools.partial(_sc_kernel_body, B),
        mesh=mesh,
        out_type=jax.ShapeDtypeStruct((B * MAXLEN * D_MODEL,), jnp.float32),
        scratch_types=(
            [pltpu.VMEM((CHW,), jnp.float32)] * NBUF
            + [pltpu.SemaphoreType.DMA] * (2 * NBUF)
        ),
    )


def kernel(x, pos_table):
    B, S, D = x.shape
    xf = jnp.reshape(x, (B * S * D,))
    pf = jnp.reshape(pos_table, (S * D,))
    out = _make_sc_call(B)(xf, pf)
    return jnp.reshape(out, (B, S, D))


# speedup vs baseline: 3.1694x; 3.1694x over previous
import jax
import jax.numpy as jnp
from jax.experimental import pallas as pl
from jax.experimental.pallas import tpu as pltpu

BM = 256


def _add_kernel(x_ref, pos_ref, out_ref):
    out_ref[...] = x_ref[...] + pos_ref[...]


def kernel(x, pos_table):
    B, S, D = x.shape
    grid = (S // BM, B)
    out = pl.pallas_call(
        _add_kernel,
        grid=grid,
        in_specs=[
            pl.BlockSpec((1, BM, D), lambda i, b: (b, i, 0)),
            pl.BlockSpec((BM, D), lambda i, b: (i, 0)),
        ],
        out_specs=pl.BlockSpec((1, BM, D), lambda i, b: (b, i, 0)),
        out_shape=jax.ShapeDtypeStruct((B, S, D), x.dtype),
        compiler_params=pltpu.CompilerParams(
            dimension_semantics=("parallel", "arbitrary"),
        ),
    )(x, pos_table)
    return out
